# baseline (speedup 1.0000x reference)
import math

import jax
import jax.numpy as jnp
from jax.experimental import pallas as pl
from jax.experimental.pallas import tpu as pltpu

B = 64
Z = 1024
D = 1024
OUT = 256
_INV_SQRT_D = 1.0 / math.sqrt(D)


def _zoner_kernel(txt_ref, zone_ref, wt_ref, bt_ref, wz_ref, bz_ref, out_ref):
    t = jnp.tanh(
        jnp.dot(txt_ref[0], wt_ref[...], preferred_element_type=jnp.float32)
        + bt_ref[...]
    )
    z = jnp.tanh(
        jnp.dot(zone_ref[0], wz_ref[...], preferred_element_type=jnp.float32)
        + bz_ref[...]
    )
    logits = jax.lax.dot_general(
        t, z, (((1,), (1,)), ((), ())), preferred_element_type=jnp.float32
    ) * _INV_SQRT_D
    m = jnp.max(logits, axis=1, keepdims=True)
    e = jnp.exp(logits - m)
    out_ref[0] = e / jnp.sum(e, axis=1, keepdims=True)


def kernel(txt_embeds, zone_embeds, W_txt, b_txt, W_zone, b_zone):
    txt3 = txt_embeds.reshape(B, 1, D)
    bt = b_txt.reshape(1, OUT)
    bz = b_zone.reshape(1, OUT)
    out = pl.pallas_call(
        _zoner_kernel,
        grid=(B,),
        in_specs=[
            pl.BlockSpec((1, 1, D), lambda b: (b, 0, 0)),
            pl.BlockSpec((1, Z, D), lambda b: (b, 0, 0)),
            pl.BlockSpec((D, OUT), lambda b: (0, 0)),
            pl.BlockSpec((1, OUT), lambda b: (0, 0)),
            pl.BlockSpec((D, OUT), lambda b: (0, 0)),
            pl.BlockSpec((1, OUT), lambda b: (0, 0)),
        ],
        out_specs=pl.BlockSpec((1, 1, Z), lambda b: (b, 0, 0)),
        out_shape=jax.ShapeDtypeStruct((B, 1, Z), jnp.float32),
        compiler_params=pltpu.CompilerParams(
            dimension_semantics=("arbitrary",),
        ),
    )(txt3, zone_embeds, W_txt, bt, W_zone, bz)
    return out.reshape(B, Z)
